# baseline (speedup 1.0000x reference)
import functools

import jax
import jax.numpy as jnp
from jax.experimental import pallas as pl

B, T, D = 2, 2048, 2048
E = 8
O = 2048
I = D // E
C = 256
TB = 256
NT = T // TB


def _ws_kernel(w_ref, b_ref, ws_ref, bs_ref):
    wblk = w_ref[...]
    acc = wblk[:, 0:I]
    for j in range(1, E):
        acc = acc + wblk[:, j * I:(j + 1) * I]
    ws_ref[...] = jnp.sum(acc, axis=0, keepdims=True).reshape(1, 1, I)

    e = pl.program_id(0)

    @pl.when(e == 0)
    def _():
        bs_ref[...] = jnp.sum(b_ref[...]).reshape(1, 1)


def _ps_kernel(x_ref, dm_ref, ws_ref, bs_ref, s_ref):
    t = pl.program_id(1)

    @pl.when(t == 0)
    def _():
        s_ref[...] = jnp.full_like(s_ref, bs_ref[0, 0])

    xb = x_ref[0]
    dmb = dm_ref[0]
    prod = xb * ws_ref[...]
    parts = []
    for e in range(E):
        p_e = jnp.sum(prod[:, e * I:(e + 1) * I], axis=1, keepdims=True)
        contrib = jnp.sum(dmb[:, e * C:(e + 1) * C] * p_e, axis=0,
                          keepdims=True)
        parts.append(contrib)
    s_ref[...] += jnp.concatenate(parts, axis=1).reshape(1, 1, E * C)


def _out_kernel(cb_ref, s_ref, o_ref):
    prod = cb_ref[0] * s_ref[0]
    o_ref[...] = jnp.sum(prod, axis=1).reshape(1, 1, TB)


def kernel(x, combine_array, dispatch_mask, W, b):
    dm2 = dispatch_mask.reshape(B, T, E * C)
    cb2 = combine_array.reshape(B, T, E * C)
    b2 = b.reshape(E, O // E)

    ws, bs = pl.pallas_call(
        _ws_kernel,
        grid=(E,),
        in_specs=[
            pl.BlockSpec((O // E, D), lambda e: (e, 0)),
            pl.BlockSpec((E, O // E), lambda e: (0, 0)),
        ],
        out_specs=[
            pl.BlockSpec((1, 1, I), lambda e: (e, 0, 0)),
            pl.BlockSpec((1, 1), lambda e: (0, 0)),
        ],
        out_shape=[
            jax.ShapeDtypeStruct((E, 1, I), jnp.float32),
            jax.ShapeDtypeStruct((1, 1), jnp.float32),
        ],
    )(W, b2)

    wsf = ws.reshape(1, D)

    s = pl.pallas_call(
        _ps_kernel,
        grid=(B, NT),
        in_specs=[
            pl.BlockSpec((1, TB, D), lambda bb, t: (bb, t, 0)),
            pl.BlockSpec((1, TB, E * C), lambda bb, t: (bb, t, 0)),
            pl.BlockSpec((1, D), lambda bb, t: (0, 0)),
            pl.BlockSpec((1, 1), lambda bb, t: (0, 0)),
        ],
        out_specs=pl.BlockSpec((1, 1, E * C), lambda bb, t: (bb, 0, 0)),
        out_shape=jax.ShapeDtypeStruct((B, 1, E * C), jnp.float32),
    )(x, dm2, wsf, bs)

    out = pl.pallas_call(
        _out_kernel,
        grid=(B, NT),
        in_specs=[
            pl.BlockSpec((1, TB, E * C), lambda bb, t: (bb, t, 0)),
            pl.BlockSpec((1, 1, E * C), lambda bb, t: (bb, 0, 0)),
        ],
        out_specs=pl.BlockSpec((1, 1, TB), lambda bb, t: (bb, 0, t)),
        out_shape=jax.ShapeDtypeStruct((B, 1, T), jnp.float32),
    )(cb2, s)

    return out.reshape(B, T)


# speedup vs baseline: 1.0448x; 1.0448x over previous
import functools

import jax
import jax.numpy as jnp
from jax.experimental import pallas as pl

B, T, D = 2, 2048, 2048
E = 8
O = 2048
I = D // E
C = 256
TB = 1024
NT = T // TB


def _ws_kernel(w_ref, b_ref, ws_ref, bs_ref):
    wblk = w_ref[...]
    acc = wblk[:, 0:I]
    for j in range(1, E):
        acc = acc + wblk[:, j * I:(j + 1) * I]
    ws_ref[...] = jnp.sum(acc, axis=0, keepdims=True).reshape(1, 1, I)

    e = pl.program_id(0)

    @pl.when(e == 0)
    def _():
        bs_ref[...] = jnp.sum(b_ref[...]).reshape(1, 1)


def _ps_kernel(x_ref, dm_ref, ws_ref, bs_ref, s_ref):
    t = pl.program_id(1)

    @pl.when(t == 0)
    def _():
        s_ref[...] = jnp.full_like(s_ref, bs_ref[0, 0])

    xb = x_ref[0]
    dmb = dm_ref[0]
    prod = xb * ws_ref[...]
    parts = []
    for e in range(E):
        p_e = jnp.sum(prod[:, e * I:(e + 1) * I], axis=1, keepdims=True)
        contrib = jnp.sum(dmb[:, e * C:(e + 1) * C] * p_e, axis=0,
                          keepdims=True)
        parts.append(contrib)
    s_ref[...] += jnp.concatenate(parts, axis=1).reshape(1, 1, E * C)


def _out_kernel(cb_ref, s_ref, o_ref):
    prod = cb_ref[0] * s_ref[0]
    o_ref[...] = jnp.sum(prod, axis=1).reshape(1, 1, TB)


def kernel(x, combine_array, dispatch_mask, W, b):
    dm2 = dispatch_mask.reshape(B, T, E * C)
    cb2 = combine_array.reshape(B, T, E * C)
    b2 = b.reshape(E, O // E)

    ws, bs = pl.pallas_call(
        _ws_kernel,
        grid=(E,),
        in_specs=[
            pl.BlockSpec((O // E, D), lambda e: (e, 0)),
            pl.BlockSpec((E, O // E), lambda e: (0, 0)),
        ],
        out_specs=[
            pl.BlockSpec((1, 1, I), lambda e: (e, 0, 0)),
            pl.BlockSpec((1, 1), lambda e: (0, 0)),
        ],
        out_shape=[
            jax.ShapeDtypeStruct((E, 1, I), jnp.float32),
            jax.ShapeDtypeStruct((1, 1), jnp.float32),
        ],
    )(W, b2)

    wsf = ws.reshape(1, D)

    s = pl.pallas_call(
        _ps_kernel,
        grid=(B, NT),
        in_specs=[
            pl.BlockSpec((1, TB, D), lambda bb, t: (bb, t, 0)),
            pl.BlockSpec((1, TB, E * C), lambda bb, t: (bb, t, 0)),
            pl.BlockSpec((1, D), lambda bb, t: (0, 0)),
            pl.BlockSpec((1, 1), lambda bb, t: (0, 0)),
        ],
        out_specs=pl.BlockSpec((1, 1, E * C), lambda bb, t: (bb, 0, 0)),
        out_shape=jax.ShapeDtypeStruct((B, 1, E * C), jnp.float32),
    )(x, dm2, wsf, bs)

    out = pl.pallas_call(
        _out_kernel,
        grid=(B, NT),
        in_specs=[
            pl.BlockSpec((1, TB, E * C), lambda bb, t: (bb, t, 0)),
            pl.BlockSpec((1, 1, E * C), lambda bb, t: (bb, 0, 0)),
        ],
        out_specs=pl.BlockSpec((1, 1, TB), lambda bb, t: (bb, 0, t)),
        out_shape=jax.ShapeDtypeStruct((B, 1, T), jnp.float32),
    )(cb2, s)

    return out.reshape(B, T)
